# baseline (speedup 1.0000x reference)
import functools

import jax
import jax.numpy as jnp
from jax import lax
from jax.experimental import pallas as pl
from jax.experimental.pallas import tpu as pltpu
from jax.experimental.pallas import tpu_sc as plsc

_NC, _NS, _L = 2, 16, 16


def _make_sc_add(rows, seq, d):
    nw = _NC * _NS
    rows_per_w = rows // nw
    c_rows = 32
    nchunk = rows_per_w // c_rows
    cn = c_rows * d

    mesh = plsc.VectorSubcoreMesh(core_axis_name="c", subcore_axis_name="s")

    def body(x_hbm, pos_hbm, out_hbm, xbuf, pbuf):
        wid = lax.axis_index("s") * _NC + lax.axis_index("c")
        base = wid * rows_per_w
        pbase = lax.rem(base, seq)
        for k in range(nchunk):
            off = (base + k * c_rows) * d
            poff = (pbase + k * c_rows) * d
            pltpu.sync_copy(x_hbm.at[pl.ds(off, cn)], xbuf)
            pltpu.sync_copy(pos_hbm.at[pl.ds(poff, cn)], pbuf)

            @plsc.parallel_loop(0, cn, step=_L, unroll=8)
            def _(i):
                plsc.addupdate(xbuf.at[pl.ds(i, _L)], pbuf[pl.ds(i, _L)])

            pltpu.sync_copy(xbuf, out_hbm.at[pl.ds(off, cn)])

    return pl.kernel(
        body,
        out_type=jax.ShapeDtypeStruct((rows * d,), jnp.float32),
        mesh=mesh,
        scratch_types=[
            pltpu.VMEM((cn,), jnp.float32),
            pltpu.VMEM((cn,), jnp.float32),
        ],
    )


def kernel(x, pos_embedding):
    B, S, D = x.shape
    xf = x.reshape(B * S * D)
    pf = pos_embedding[:S].reshape(S * D)
    out = _make_sc_add(B * S, S, D)(xf, pf)
    return out.reshape(B, S, D)


# speedup vs baseline: 1.1777x; 1.1777x over previous
import functools

import jax
import jax.numpy as jnp
from jax import lax
from jax.experimental import pallas as pl
from jax.experimental.pallas import tpu as pltpu
from jax.experimental.pallas import tpu_sc as plsc

_NC, _NS, _L = 2, 16, 16


def _make_sc_add(rows, seq, d):
    nw = _NC * _NS
    rows_per_w = rows // nw
    c_rows = 16
    nchunk = rows_per_w // c_rows
    cn = c_rows * d

    mesh = plsc.VectorSubcoreMesh(core_axis_name="c", subcore_axis_name="s")

    def body(x_hbm, pos_hbm, out_hbm, xb, pb, ob, sx, sp, so):
        wid = lax.axis_index("s") * _NC + lax.axis_index("c")
        base = wid * rows_per_w
        pbase = lax.rem(base, seq)

        def start_in(k, b):
            off = (base + k * c_rows) * d
            poff = (pbase + k * c_rows) * d
            hx = pltpu.async_copy(x_hbm.at[pl.ds(off, cn)], xb[b], sx[b])
            hp = pltpu.async_copy(pos_hbm.at[pl.ds(poff, cn)], pb[b], sp[b])
            return hx, hp

        hin = [start_in(0, 0), start_in(1, 1)]
        hout = [None, None]
        for k in range(nchunk):
            b = k & 1
            hin[b][0].wait()
            hin[b][1].wait()
            if hout[b] is not None:
                hout[b].wait()

            @plsc.parallel_loop(0, cn, step=_L, unroll=8)
            def _(i):
                ob[b][pl.ds(i, _L)] = xb[b][pl.ds(i, _L)] + pb[b][pl.ds(i, _L)]

            off = (base + k * c_rows) * d
            hout[b] = pltpu.async_copy(ob[b], out_hbm.at[pl.ds(off, cn)], so[b])
            if k + 2 < nchunk:
                hin[b] = start_in(k + 2, b)
        for h in hout:
            if h is not None:
                h.wait()

    buf = lambda: [pltpu.VMEM((cn,), jnp.float32) for _ in range(2)]
    sem = lambda: [pltpu.SemaphoreType.DMA for _ in range(2)]
    return pl.kernel(
        body,
        out_type=jax.ShapeDtypeStruct((rows * d,), jnp.float32),
        mesh=mesh,
        scratch_types=[buf(), buf(), buf(), sem(), sem(), sem()],
    )


def kernel(x, pos_embedding):
    B, S, D = x.shape
    xf = x.reshape(B * S * D)
    pf = pos_embedding[:S].reshape(S * D)
    out = _make_sc_add(B * S, S, D)(xf, pf)
    return out.reshape(B, S, D)


# speedup vs baseline: 2.9991x; 2.5465x over previous
import functools

import jax
import jax.numpy as jnp
from jax import lax
from jax.experimental import pallas as pl
from jax.experimental.pallas import tpu as pltpu
from jax.experimental.pallas import tpu_sc as plsc

_NC, _NS, _L = 2, 16, 16


def _make_sc_add(nb, seq, d):
    nw = _NC * _NS
    pos_per_w = seq // nw
    c_rows = 16
    nchunk = pos_per_w // c_rows

    mesh = plsc.VectorSubcoreMesh(core_axis_name="c", subcore_axis_name="s")

    def body(x_hbm, pos_hbm, out_hbm, xb, pb, ob, sx, sp, so):
        wid = lax.axis_index("s") * _NC + lax.axis_index("c")
        wbase = wid * pos_per_w
        pairs = [(k, b) for k in range(nchunk) for b in range(nb)]

        def start_pos(k):
            return pltpu.async_copy(
                pos_hbm.at[pl.ds(wbase + k * c_rows, c_rows), :], pb[k & 1], sp[k & 1]
            )

        def start_x(t, bb):
            k, b = pairs[t]
            row = b * seq + wbase + k * c_rows
            return pltpu.async_copy(x_hbm.at[pl.ds(row, c_rows), :], xb[bb], sx[bb])

        hp = [start_pos(0), None]
        hx = [start_x(0, 0), start_x(1, 1)]
        hout = [None, None]
        for t, (k, b) in enumerate(pairs):
            bb = t & 1
            kk = k & 1
            if b == 0:
                hp[kk].wait()
            hx[bb].wait()
            if hout[bb] is not None:
                hout[bb].wait()

            @plsc.parallel_loop(0, c_rows)
            def _(r):
                @plsc.parallel_loop(0, d, step=_L, unroll=8)
                def _(i):
                    ob[bb][r, pl.ds(i, _L)] = xb[bb][r, pl.ds(i, _L)] + pb[kk][r, pl.ds(i, _L)]

            row = b * seq + wbase + k * c_rows
            hout[bb] = pltpu.async_copy(ob[bb], out_hbm.at[pl.ds(row, c_rows), :], so[bb])
            if t + 2 < len(pairs):
                hx[bb] = start_x(t + 2, bb)
            if b == 0 and k + 1 < nchunk:
                hp[(k + 1) & 1] = start_pos(k + 1)
        for h in hout:
            if h is not None:
                h.wait()

    buf = lambda: [pltpu.VMEM((c_rows, d), jnp.float32) for _ in range(2)]
    sem = lambda: [pltpu.SemaphoreType.DMA for _ in range(2)]
    return pl.kernel(
        body,
        out_type=jax.ShapeDtypeStruct((nb * seq, d), jnp.float32),
        mesh=mesh,
        scratch_types=[buf(), buf(), buf(), sem(), sem(), sem()],
    )


def kernel(x, pos_embedding):
    B, S, D = x.shape
    x2 = x.reshape(B * S, D)
    out = _make_sc_add(B, S, D)(x2, pos_embedding[:S])
    return out.reshape(B, S, D)
